# baseline (speedup 1.0000x reference)
import functools

import jax
import jax.numpy as jnp
from jax import lax
from jax.experimental import pallas as pl
from jax.experimental.pallas import tpu as pltpu
from jax.experimental.pallas import tpu_sc as plsc

N_NODES = 10000
N_EDGES = 320000

NC = 2
NS = 16
NW = NC * NS
EW = N_EDGES // NW
K = 125
CH = EW // K
RPT = 624
REM = N_NODES - NS * RPT

_mesh = functools.partial(
    plsc.VectorSubcoreMesh, core_axis_name="c", subcore_axis_name="s")


def _wid():
    return lax.axis_index("s") * NC + lax.axis_index("c")


def _al8(i):
    return pl.multiple_of(i, 8)


@functools.partial(
    pl.kernel,
    out_type=[
        jax.ShapeDtypeStruct((NC, N_NODES, 16), jnp.float32),
        jax.ShapeDtypeStruct((NC, N_NODES, 16), jnp.float32),
    ],
    mesh=_mesh(),
    compiler_params=pltpu.CompilerParams(use_tc_tiling_on_sc=False),
    scratch_types=[
        pltpu.VMEM((2 * CH, K), jnp.int32),
        pltpu.VMEM((K, 16), jnp.float32),
        pltpu.VMEM_SHARED((N_NODES, 16), jnp.float32),
        pltpu.VMEM_SHARED((N_NODES, 16), jnp.float32),
        pltpu.SemaphoreType.DMA,
    ],
)
def _sc_degrees(sd2d,
                dsrc_out, ddst_out,
                sd_v, ones_v, acc_s, acc_d, dsem):
    c = lax.axis_index("c")
    s = lax.axis_index("s")
    w = _wid()
    pltpu.sync_copy(sd2d.at[pl.ds(_al8(2 * w * CH), 2 * CH)], sd_v)

    z16 = jnp.zeros((16,), jnp.float32)

    @pl.loop(0, K)
    def _(i):
        ones_v[i, :] = z16

    for m in range(RPT // 104):
        zsl = pl.ds(_al8(s * RPT + m * 104), 104)
        pltpu.sync_copy(ones_v.at[pl.ds(0, 104)], acc_s.at[zsl])
        pltpu.sync_copy(ones_v.at[pl.ds(0, 104)], acc_d.at[zsl])

    @pl.when(s == 0)
    def _():
        pltpu.sync_copy(ones_v.at[pl.ds(0, REM)], acc_s.at[pl.ds(NS * RPT, REM)])
        pltpu.sync_copy(ones_v.at[pl.ds(0, REM)], acc_d.at[pl.ds(NS * RPT, REM)])

    o16 = jnp.ones((16,), jnp.float32)

    @pl.loop(0, K)
    def _(i):
        ones_v[i, :] = o16

    plsc.subcore_barrier()

    @pl.loop(0, CH)
    def _(j):
        pltpu.async_copy(ones_v, acc_s.at[sd_v.at[2 * j]], dsem, add=True)
        pltpu.async_copy(ones_v, acc_d.at[sd_v.at[2 * j + 1]], dsem, add=True)

    @pl.loop(0, 2 * CH)
    def _(j):
        pltpu.make_async_copy(ones_v, acc_s.at[sd_v.at[0]], dsem).wait()

    plsc.subcore_barrier()
    sl = pl.ds(_al8(s * RPT), RPT)
    pltpu.sync_copy(acc_s.at[sl], dsrc_out.at[c].at[sl])
    pltpu.sync_copy(acc_d.at[sl], ddst_out.at[c].at[sl])

    @pl.when(s == 0)
    def _():
        rl = pl.ds(NS * RPT, REM)
        pltpu.sync_copy(acc_s.at[rl], dsrc_out.at[c].at[rl])
        pltpu.sync_copy(acc_d.at[rl], ddst_out.at[c].at[rl])


def _make_sc_scatter(F):
    @functools.partial(
        pl.kernel,
        out_type=jax.ShapeDtypeStruct((NC, N_NODES, F), jnp.float32),
        mesh=_mesh(),
        compiler_params=pltpu.CompilerParams(use_tc_tiling_on_sc=False),
        scratch_types=[
            pltpu.VMEM((16, K), jnp.int32),
            pltpu.VMEM((CH, K), jnp.int32),
            pltpu.VMEM((K, F), jnp.float32),
            pltpu.VMEM((K, F), jnp.float32),
            pltpu.VMEM_SHARED((N_NODES, F), jnp.float32),
            pltpu.SemaphoreType.DMA,
            pltpu.SemaphoreType.DMA,
            pltpu.SemaphoreType.DMA,
        ],
    )
    def sc_scatter(src2d, dst2d, h_hbm, part_out,
                   src_v, dst_v, rows0, rows1, acc, sem0, sem1, rsem):
        c = lax.axis_index("c")
        s = lax.axis_index("s")
        w = _wid()
        pltpu.sync_copy(src2d.at[pl.ds(_al8(w * CH), 8)], src_v.at[pl.ds(0, 8)])
        pltpu.sync_copy(dst2d.at[pl.ds(_al8(w * CH), CH)], dst_v)

        z16 = jnp.zeros((16,), jnp.float32)

        @pl.loop(0, K)
        def _(i):
            for g in range(F // 16):
                rows0[i, pl.ds(16 * g, 16)] = z16

        for m in range(RPT // 104):
            pltpu.async_copy(rows0.at[pl.ds(0, 104)],
                             acc.at[pl.ds(_al8(s * RPT + m * 104), 104)], sem1)
        for m in range(RPT // 104):
            pltpu.make_async_copy(
                rows0.at[pl.ds(0, 104)],
                acc.at[pl.ds(_al8(s * RPT + m * 104), 104)], sem1).wait()

        @pl.when(s == 0)
        def _():
            pltpu.sync_copy(rows0.at[pl.ds(0, REM)],
                            acc.at[pl.ds(NS * RPT, REM)])

        plsc.subcore_barrier()

        pltpu.async_copy(h_hbm.at[src_v.at[0]], rows0, sem0)

        @pl.loop(0, CH, step=2)
        def _(j):
            @pl.when(jnp.logical_and(lax.rem(j, 8) == 0, j + 8 < CH))
            def _():
                pltpu.async_copy(
                    src2d.at[pl.ds(_al8(w * CH + j + 8), 8)],
                    src_v.at[pl.ds(_al8(lax.rem(j // 8 + 1, 2) * 8), 8)],
                    rsem)

            @pl.when(jnp.logical_and(lax.rem(j, 8) == 6, j + 2 < CH))
            def _():
                pltpu.make_async_copy(
                    src2d.at[pl.ds(_al8(w * CH), 8)],
                    src_v.at[pl.ds(0, 8)], rsem).wait()

            pltpu.async_copy(h_hbm.at[src_v.at[lax.rem(j + 1, 16)]], rows1, sem1)
            pltpu.make_async_copy(h_hbm.at[src_v.at[lax.rem(j, 16)]],
                                  rows0, sem0).wait()
            pltpu.sync_copy(rows0, acc.at[dst_v.at[j]], add=True)

            @pl.when(j + 2 < CH)
            def _():
                pltpu.async_copy(h_hbm.at[src_v.at[lax.rem(j + 2, 16)]],
                                 rows0, sem0)

            pltpu.make_async_copy(h_hbm.at[src_v.at[lax.rem(j + 1, 16)]],
                                  rows1, sem1).wait()
            pltpu.sync_copy(rows1, acc.at[dst_v.at[j + 1]], add=True)

        plsc.subcore_barrier()
        sl = pl.ds(_al8(s * RPT), RPT)
        pltpu.sync_copy(acc.at[sl], part_out.at[c].at[sl])

        @pl.when(s == 0)
        def _():
            rl = pl.ds(NS * RPT, REM)
            pltpu.sync_copy(acc.at[rl], part_out.at[c].at[rl])

    return sc_scatter


def _make_sc_scatter_nbuf(F, B, D):
    @functools.partial(
        pl.kernel,
        out_type=jax.ShapeDtypeStruct((NC, N_NODES, F), jnp.float32),
        mesh=_mesh(),
        compiler_params=pltpu.CompilerParams(use_tc_tiling_on_sc=False),
        scratch_types=(
            [pltpu.VMEM((2 * CH, K), jnp.int32)]
            + [pltpu.VMEM((K, F), jnp.float32) for _ in range(B)]
            + [pltpu.VMEM_SHARED((N_NODES, F), jnp.float32)]
            + [pltpu.SemaphoreType.DMA for _ in range(2 * B)]
        ),
    )
    def sc_scatter(sd2d, h_hbm, part_out, sd_v, *bufs_sems):
        rows = bufs_sems[:B]
        acc = bufs_sems[B]
        gsem = bufs_sems[B + 1:2 * B + 1]
        ssem = bufs_sems[2 * B + 1:3 * B + 1]
        c = lax.axis_index("c")
        s = lax.axis_index("s")
        w = _wid()
        pltpu.sync_copy(sd2d.at[pl.ds(_al8(2 * w * CH), 2 * CH)], sd_v)

        z16 = jnp.zeros((16,), jnp.float32)

        @pl.loop(0, K)
        def _(i):
            for g in range(F // 16):
                rows[0][i, pl.ds(16 * g, 16)] = z16

        for m in range(RPT // 104):
            pltpu.async_copy(rows[0].at[pl.ds(0, 104)],
                             acc.at[pl.ds(_al8(s * RPT + m * 104), 104)], ssem[0])
        for m in range(RPT // 104):
            pltpu.make_async_copy(rows[0].at[pl.ds(0, 104)],
                                  acc.at[pl.ds(_al8(s * RPT + m * 104), 104)], ssem[0]).wait()

        @pl.when(s == 0)
        def _():
            pltpu.sync_copy(rows[0].at[pl.ds(0, REM)],
                            acc.at[pl.ds(NS * RPT, REM)])

        plsc.subcore_barrier()

        for d in range(D):
            pltpu.async_copy(h_hbm.at[sd_v.at[2 * d]], rows[d], gsem[d])

        @pl.loop(0, CH, step=B)
        def _(j):
            for b in range(B):
                jj = j + b
                bn = (b + D) % B

                @pl.when(jj + D < CH)
                def _():
                    @pl.when(jj + D >= B)
                    def _():
                        pltpu.make_async_copy(
                            rows[bn], acc.at[sd_v.at[1]], ssem[bn]).wait()

                    pltpu.async_copy(
                        h_hbm.at[sd_v.at[2 * (jj + D)]], rows[bn], gsem[bn])

                pltpu.make_async_copy(
                    h_hbm.at[sd_v.at[2 * jj]], rows[b], gsem[b]).wait()
                pltpu.async_copy(rows[b], acc.at[sd_v.at[2 * jj + 1]],
                                 ssem[b], add=True)

        for cc in range(CH - B, CH):
            pltpu.make_async_copy(rows[cc % B], acc.at[sd_v.at[1]],
                                  ssem[cc % B]).wait()

        plsc.subcore_barrier()
        sl = pl.ds(_al8(s * RPT), RPT)
        pltpu.sync_copy(acc.at[sl], part_out.at[c].at[sl])

        @pl.when(s == 0)
        def _():
            rl = pl.ds(NS * RPT, REM)
            pltpu.sync_copy(acc.at[rl], part_out.at[c].at[rl])

    return sc_scatter


def _make_sc_scatter128_b3():
    F, B, D = 128, 3, 1

    def r_src(cc):
        return lax.rem(cc // 4, 2) * 8 + 2 * lax.rem(cc, 4)

    @functools.partial(
        pl.kernel,
        out_type=jax.ShapeDtypeStruct((NC, N_NODES, F), jnp.float32),
        mesh=_mesh(),
        compiler_params=pltpu.CompilerParams(use_tc_tiling_on_sc=False),
        scratch_types=(
            [pltpu.VMEM((16, K), jnp.int32)]
            + [pltpu.VMEM((K, F), jnp.float32) for _ in range(B)]
            + [pltpu.VMEM_SHARED((N_NODES, F), jnp.float32)]
            + [pltpu.SemaphoreType.DMA for _ in range(2 * B + 1)]
        ),
    )
    def sc_scatter(sd2d, h_hbm, part_out, idx_v, *bufs_sems):
        rows = bufs_sems[:B]
        acc = bufs_sems[B]
        gsem = bufs_sems[B + 1:2 * B + 1]
        ssem = bufs_sems[2 * B + 1:3 * B + 1]
        rsem = bufs_sems[3 * B + 1]
        c = lax.axis_index("c")
        s = lax.axis_index("s")
        w = _wid()
        pltpu.sync_copy(sd2d.at[pl.ds(_al8(2 * w * CH), 16)], idx_v)

        z16 = jnp.zeros((16,), jnp.float32)

        @pl.loop(0, K)
        def _(i):
            for g in range(F // 16):
                rows[0][i, pl.ds(16 * g, 16)] = z16

        for m in range(RPT // 104):
            pltpu.async_copy(rows[0].at[pl.ds(0, 104)],
                             acc.at[pl.ds(_al8(s * RPT + m * 104), 104)],
                             ssem[0])
        for m in range(RPT // 104):
            pltpu.make_async_copy(rows[0].at[pl.ds(0, 104)],
                                  acc.at[pl.ds(_al8(s * RPT + m * 104), 104)],
                                  ssem[0]).wait()

        @pl.when(s == 0)
        def _():
            pltpu.sync_copy(rows[0].at[pl.ds(0, REM)],
                            acc.at[pl.ds(NS * RPT, REM)])

        plsc.subcore_barrier()

        pltpu.async_copy(h_hbm.at[idx_v.at[0]], rows[0], gsem[0])

        @pl.loop(0, CH - CH % B, step=B)
        def _(j):
            for b in range(B):
                jj = j + b
                bn = (b + D) % B

                @pl.when(jnp.logical_and(
                        jnp.logical_and(lax.rem(jj, 4) == 2, jj >= 6),
                        (jj // 4 + 1) * 4 < CH))
                def _():
                    pltpu.async_copy(
                        sd2d.at[pl.ds(_al8(2 * (w * CH + (jj // 4 + 1) * 4)),
                                      8)],
                        idx_v.at[pl.ds(_al8(lax.rem(jj // 4 + 1, 2) * 8), 8)],
                        rsem)

                @pl.when(jnp.logical_and(
                        jnp.logical_and(lax.rem(jj, 4) == 3, jj >= 7),
                        jj + 1 < CH))
                def _():
                    pltpu.make_async_copy(
                        sd2d.at[pl.ds(_al8(2 * w * CH), 8)],
                        idx_v.at[pl.ds(0, 8)], rsem).wait()

                @pl.when(jj + D < CH)
                def _():
                    @pl.when(jj + D >= B)
                    def _():
                        pltpu.make_async_copy(
                            rows[bn], acc.at[idx_v.at[1]], ssem[bn]).wait()

                    pltpu.async_copy(
                        h_hbm.at[idx_v.at[r_src(jj + D)]], rows[bn],
                        gsem[bn])

                pltpu.make_async_copy(
                    h_hbm.at[idx_v.at[r_src(jj)]], rows[b], gsem[b]).wait()
                pltpu.async_copy(rows[b], acc.at[idx_v.at[r_src(jj) + 1]],
                                 ssem[b], add=True)

        pltpu.make_async_copy(rows[1], acc.at[idx_v.at[1]], ssem[1]).wait()
        pltpu.async_copy(h_hbm.at[idx_v.at[14]], rows[1], gsem[1])
        pltpu.make_async_copy(h_hbm.at[idx_v.at[12]], rows[0], gsem[0]).wait()
        pltpu.async_copy(rows[0], acc.at[idx_v.at[13]], ssem[0], add=True)
        pltpu.make_async_copy(h_hbm.at[idx_v.at[14]], rows[1], gsem[1]).wait()
        pltpu.async_copy(rows[1], acc.at[idx_v.at[15]], ssem[1], add=True)
        for b in range(B):
            pltpu.make_async_copy(rows[b], acc.at[idx_v.at[1]],
                                  ssem[b]).wait()

        plsc.subcore_barrier()
        sl = pl.ds(_al8(s * RPT), RPT)
        pltpu.sync_copy(acc.at[sl], part_out.at[c].at[sl])

        @pl.when(s == 0)
        def _():
            rl = pl.ds(NS * RPT, REM)
            pltpu.sync_copy(acc.at[rl], part_out.at[c].at[rl])

    return sc_scatter


_sc_scatter128 = _make_sc_scatter128_b3()
_sc_scatter64 = _make_sc_scatter_nbuf(64, 4, 2)


BN = 1000


def _tc_mm0(features, W0):
    def body(x_ref, w_ref, h_ref):
        h_ref[...] = jnp.dot(x_ref[...], w_ref[...],
                             preferred_element_type=jnp.float32)

    return pl.pallas_call(
        body,
        grid=(N_NODES // BN,),
        in_specs=[
            pl.BlockSpec((BN, 128), lambda i: (i, 0)),
            pl.BlockSpec((128, 128), lambda i: (0, 0)),
        ],
        out_specs=pl.BlockSpec((BN, 128), lambda i: (i, 0)),
        out_shape=jax.ShapeDtypeStruct((N_NODES, 128), jnp.float32),
    )(features, W0)


def _tc_norms(h0, dsrc_p, ddst_p):
    def body(h0_ref, ds_ref, dd_ref, h_ref, ns_ref, nd_ref):
        ds = ds_ref[0, :, 0:1] + ds_ref[1, :, 0:1]
        dd = dd_ref[0, :, 0:1] + dd_ref[1, :, 0:1]
        ns = lax.rsqrt(jnp.maximum(ds, 1.0))
        nd = lax.rsqrt(jnp.maximum(dd, 1.0))
        ns_ref[...] = ns
        nd_ref[...] = nd
        h_ref[...] = h0_ref[...] * ns

    return pl.pallas_call(
        body,
        grid=(N_NODES // BN,),
        in_specs=[
            pl.BlockSpec((BN, 128), lambda i: (i, 0)),
            pl.BlockSpec((NC, BN, 16), lambda i: (0, i, 0)),
            pl.BlockSpec((NC, BN, 16), lambda i: (0, i, 0)),
        ],
        out_specs=[
            pl.BlockSpec((BN, 128), lambda i: (i, 0)),
            pl.BlockSpec((BN, 1), lambda i: (i, 0)),
            pl.BlockSpec((BN, 1), lambda i: (i, 0)),
        ],
        out_shape=[
            jax.ShapeDtypeStruct((N_NODES, 128), jnp.float32),
            jax.ShapeDtypeStruct((N_NODES, 1), jnp.float32),
            jax.ShapeDtypeStruct((N_NODES, 1), jnp.float32),
        ],
    )(h0, dsrc_p, ddst_p)


def _make_tc_combine(F, FO, act, emit_agg, with_mm):
    def body(*refs):
        if with_mm:
            p_ref, nd_ref, b_ref, ns_ref, w_ref = refs[:5]
            orefs = refs[5:]
        else:
            p_ref, nd_ref, b_ref = refs[:3]
            orefs = refs[3:]
        agg = (p_ref[0] + p_ref[1]) * nd_ref[...] + b_ref[...]
        if act:
            agg = jnp.maximum(agg, 0.0)
        oi = 0
        if emit_agg:
            orefs[oi][...] = agg
            oi += 1
        if with_mm:
            orefs[oi][...] = jnp.dot(
                agg, w_ref[...], preferred_element_type=jnp.float32) * ns_ref[...]

    in_specs = [
        pl.BlockSpec((NC, BN, F), lambda i: (0, i, 0)),
        pl.BlockSpec((BN, 1), lambda i: (i, 0)),
        pl.BlockSpec((1, F), lambda i: (0, 0)),
    ]
    out_specs, out_shape = [], []
    if with_mm:
        in_specs += [
            pl.BlockSpec((BN, 1), lambda i: (i, 0)),
            pl.BlockSpec((F, FO), lambda i: (0, 0)),
        ]
    if emit_agg:
        out_specs.append(pl.BlockSpec((BN, F), lambda i: (i, 0)))
        out_shape.append(jax.ShapeDtypeStruct((N_NODES, F), jnp.float32))
    if with_mm:
        out_specs.append(pl.BlockSpec((BN, FO), lambda i: (i, 0)))
        out_shape.append(jax.ShapeDtypeStruct((N_NODES, FO), jnp.float32))

    kern = pl.pallas_call(
        body, grid=(N_NODES // BN,),
        in_specs=in_specs, out_specs=out_specs, out_shape=out_shape)

    def run(p, ndst, b2d, nsrc=None, W=None):
        args = (p, ndst, b2d) + ((nsrc, W) if with_mm else ())
        out = kern(*args)
        return out if len(out) > 1 else out[0]
    return run


_tc_mid_128_128 = _make_tc_combine(128, 128, act=True, emit_agg=False, with_mm=True)
_tc_mid_128_64 = _make_tc_combine(128, 64, act=True, emit_agg=False, with_mm=True)
_tc_aspect = _make_tc_combine(64, 64, act=False, emit_agg=True, with_mm=True)
_tc_last = _make_tc_combine(64, 0, act=False, emit_agg=True, with_mm=False)


def kernel(features, edge_index, W0, b0, W1, b1, W2, b2, W3, b3):
    sd2d = jnp.stack([edge_index[0].reshape(NW * CH, K),
                      edge_index[1].reshape(NW * CH, K)],
                     axis=1).reshape(2 * NW * CH, K)

    h0 = _tc_mm0(features, W0)
    dsrc_p, ddst_p = _sc_degrees(sd2d)
    h0s, nsrc, ndst = _tc_norms(h0, dsrc_p, ddst_p)

    p1 = _sc_scatter128(sd2d, h0s)
    h1s = _tc_mid_128_128(p1, ndst, b0.reshape(1, -1), nsrc, W1)

    p2 = _sc_scatter128(sd2d, h1s)
    h2s = _tc_mid_128_64(p2, ndst, b1.reshape(1, -1), nsrc, W2)

    p3 = _sc_scatter64(sd2d, h2s)
    aspect, h3s = _tc_aspect(p3, ndst, b2.reshape(1, -1), nsrc, W3)

    p4 = _sc_scatter64(sd2d, h3s)
    out = _tc_last(p4, ndst, b3.reshape(1, -1))

    return (aspect, out)


# speedup vs baseline: 1.0139x; 1.0139x over previous
import functools

import jax
import jax.numpy as jnp
from jax import lax
from jax.experimental import pallas as pl
from jax.experimental.pallas import tpu as pltpu
from jax.experimental.pallas import tpu_sc as plsc

N_NODES = 10000
N_EDGES = 320000

NC = 2
NS = 16
NW = NC * NS
EW = N_EDGES // NW
K = 125
CH = EW // K
RPT = 624
REM = N_NODES - NS * RPT

_mesh = functools.partial(
    plsc.VectorSubcoreMesh, core_axis_name="c", subcore_axis_name="s")


def _wid():
    return lax.axis_index("s") * NC + lax.axis_index("c")


def _al8(i):
    return pl.multiple_of(i, 8)


@functools.partial(
    pl.kernel,
    out_type=[
        jax.ShapeDtypeStruct((NC, N_NODES, 16), jnp.float32),
        jax.ShapeDtypeStruct((NC, N_NODES, 16), jnp.float32),
    ],
    mesh=_mesh(),
    compiler_params=pltpu.CompilerParams(use_tc_tiling_on_sc=False),
    scratch_types=[
        pltpu.VMEM((CH, K), jnp.int32),
        pltpu.VMEM((CH, K), jnp.int32),
        pltpu.VMEM((K, 16), jnp.float32),
        pltpu.VMEM_SHARED((N_NODES, 16), jnp.float32),
        pltpu.VMEM_SHARED((N_NODES, 16), jnp.float32),
        pltpu.SemaphoreType.DMA,
    ],
)
def _sc_degrees(src2d, dst2d,
                dsrc_out, ddst_out,
                src_v, dst_v, ones_v, acc_s, acc_d, dsem):
    c = lax.axis_index("c")
    s = lax.axis_index("s")
    w = _wid()
    pltpu.sync_copy(src2d.at[pl.ds(_al8(w * CH), CH)], src_v)
    pltpu.sync_copy(dst2d.at[pl.ds(_al8(w * CH), CH)], dst_v)

    z16 = jnp.zeros((16,), jnp.float32)

    @pl.loop(0, K)
    def _(i):
        ones_v[i, :] = z16

    for m in range(RPT // 104):
        zsl = pl.ds(_al8(s * RPT + m * 104), 104)
        pltpu.sync_copy(ones_v.at[pl.ds(0, 104)], acc_s.at[zsl])
        pltpu.sync_copy(ones_v.at[pl.ds(0, 104)], acc_d.at[zsl])

    @pl.when(s == 0)
    def _():
        pltpu.sync_copy(ones_v.at[pl.ds(0, REM)], acc_s.at[pl.ds(NS * RPT, REM)])
        pltpu.sync_copy(ones_v.at[pl.ds(0, REM)], acc_d.at[pl.ds(NS * RPT, REM)])

    o16 = jnp.ones((16,), jnp.float32)

    @pl.loop(0, K)
    def _(i):
        ones_v[i, :] = o16

    plsc.subcore_barrier()

    @pl.loop(0, CH)
    def _(j):
        pltpu.async_copy(ones_v, acc_s.at[src_v.at[j]], dsem, add=True)
        pltpu.async_copy(ones_v, acc_d.at[dst_v.at[j]], dsem, add=True)

    @pl.loop(0, 2 * CH)
    def _(j):
        pltpu.make_async_copy(ones_v, acc_s.at[src_v.at[0]], dsem).wait()

    plsc.subcore_barrier()
    sl = pl.ds(_al8(s * RPT), RPT)
    pltpu.sync_copy(acc_s.at[sl], dsrc_out.at[c].at[sl])
    pltpu.sync_copy(acc_d.at[sl], ddst_out.at[c].at[sl])

    @pl.when(s == 0)
    def _():
        rl = pl.ds(NS * RPT, REM)
        pltpu.sync_copy(acc_s.at[rl], dsrc_out.at[c].at[rl])
        pltpu.sync_copy(acc_d.at[rl], ddst_out.at[c].at[rl])


def _make_sc_scatter(F):
    @functools.partial(
        pl.kernel,
        out_type=jax.ShapeDtypeStruct((NC, N_NODES, F), jnp.float32),
        mesh=_mesh(),
        compiler_params=pltpu.CompilerParams(use_tc_tiling_on_sc=False),
        scratch_types=[
            pltpu.VMEM((16, K), jnp.int32),
            pltpu.VMEM((CH, K), jnp.int32),
            pltpu.VMEM((K, F), jnp.float32),
            pltpu.VMEM((K, F), jnp.float32),
            pltpu.VMEM_SHARED((N_NODES, F), jnp.float32),
            pltpu.SemaphoreType.DMA,
            pltpu.SemaphoreType.DMA,
            pltpu.SemaphoreType.DMA,
        ],
    )
    def sc_scatter(src2d, dst2d, h_hbm, part_out,
                   src_v, dst_v, rows0, rows1, acc, sem0, sem1, rsem):
        c = lax.axis_index("c")
        s = lax.axis_index("s")
        w = _wid()
        pltpu.sync_copy(src2d.at[pl.ds(_al8(w * CH), 8)], src_v.at[pl.ds(0, 8)])
        pltpu.sync_copy(dst2d.at[pl.ds(_al8(w * CH), CH)], dst_v)

        z16 = jnp.zeros((16,), jnp.float32)

        @pl.loop(0, K)
        def _(i):
            for g in range(F // 16):
                rows0[i, pl.ds(16 * g, 16)] = z16

        for m in range(RPT // 104):
            pltpu.async_copy(rows0.at[pl.ds(0, 104)],
                             acc.at[pl.ds(_al8(s * RPT + m * 104), 104)], sem1)
        for m in range(RPT // 104):
            pltpu.make_async_copy(
                rows0.at[pl.ds(0, 104)],
                acc.at[pl.ds(_al8(s * RPT + m * 104), 104)], sem1).wait()

        @pl.when(s == 0)
        def _():
            pltpu.sync_copy(rows0.at[pl.ds(0, REM)],
                            acc.at[pl.ds(NS * RPT, REM)])

        plsc.subcore_barrier()

        pltpu.async_copy(h_hbm.at[src_v.at[0]], rows0, sem0)

        @pl.loop(0, CH, step=2)
        def _(j):
            @pl.when(jnp.logical_and(lax.rem(j, 8) == 0, j + 8 < CH))
            def _():
                pltpu.async_copy(
                    src2d.at[pl.ds(_al8(w * CH + j + 8), 8)],
                    src_v.at[pl.ds(_al8(lax.rem(j // 8 + 1, 2) * 8), 8)],
                    rsem)

            @pl.when(jnp.logical_and(lax.rem(j, 8) == 6, j + 2 < CH))
            def _():
                pltpu.make_async_copy(
                    src2d.at[pl.ds(_al8(w * CH), 8)],
                    src_v.at[pl.ds(0, 8)], rsem).wait()

            pltpu.async_copy(h_hbm.at[src_v.at[lax.rem(j + 1, 16)]], rows1, sem1)
            pltpu.make_async_copy(h_hbm.at[src_v.at[lax.rem(j, 16)]],
                                  rows0, sem0).wait()
            pltpu.sync_copy(rows0, acc.at[dst_v.at[j]], add=True)

            @pl.when(j + 2 < CH)
            def _():
                pltpu.async_copy(h_hbm.at[src_v.at[lax.rem(j + 2, 16)]],
                                 rows0, sem0)

            pltpu.make_async_copy(h_hbm.at[src_v.at[lax.rem(j + 1, 16)]],
                                  rows1, sem1).wait()
            pltpu.sync_copy(rows1, acc.at[dst_v.at[j + 1]], add=True)

        plsc.subcore_barrier()
        sl = pl.ds(_al8(s * RPT), RPT)
        pltpu.sync_copy(acc.at[sl], part_out.at[c].at[sl])

        @pl.when(s == 0)
        def _():
            rl = pl.ds(NS * RPT, REM)
            pltpu.sync_copy(acc.at[rl], part_out.at[c].at[rl])

    return sc_scatter


def _make_sc_scatter_nbuf(F, B, D):
    @functools.partial(
        pl.kernel,
        out_type=jax.ShapeDtypeStruct((NC, N_NODES, F), jnp.float32),
        mesh=_mesh(),
        compiler_params=pltpu.CompilerParams(use_tc_tiling_on_sc=False),
        scratch_types=(
            [pltpu.VMEM((16, K), jnp.int32),
             pltpu.VMEM((CH, K), jnp.int32)]
            + [pltpu.VMEM((K, F), jnp.float32) for _ in range(B)]
            + [pltpu.VMEM_SHARED((N_NODES, F), jnp.float32)]
            + [pltpu.SemaphoreType.DMA for _ in range(2 * B + 1)]
        ),
    )
    def sc_scatter(src2d, dst2d, h_hbm, part_out, src_v, dst_v, *bufs_sems):
        rows = bufs_sems[:B]
        acc = bufs_sems[B]
        gsem = bufs_sems[B + 1:2 * B + 1]
        ssem = bufs_sems[2 * B + 1:3 * B + 1]
        rsem = bufs_sems[3 * B + 1]
        c = lax.axis_index("c")
        s = lax.axis_index("s")
        w = _wid()
        pltpu.sync_copy(src2d.at[pl.ds(_al8(w * CH), 8)], src_v.at[pl.ds(0, 8)])
        pltpu.sync_copy(dst2d.at[pl.ds(_al8(w * CH), CH)], dst_v)

        z16 = jnp.zeros((16,), jnp.float32)

        @pl.loop(0, K)
        def _(i):
            for g in range(F // 16):
                rows[0][i, pl.ds(16 * g, 16)] = z16

        for m in range(RPT // 104):
            pltpu.async_copy(rows[0].at[pl.ds(0, 104)],
                             acc.at[pl.ds(_al8(s * RPT + m * 104), 104)], ssem[0])
        for m in range(RPT // 104):
            pltpu.make_async_copy(rows[0].at[pl.ds(0, 104)],
                                  acc.at[pl.ds(_al8(s * RPT + m * 104), 104)], ssem[0]).wait()

        @pl.when(s == 0)
        def _():
            pltpu.sync_copy(rows[0].at[pl.ds(0, REM)],
                            acc.at[pl.ds(NS * RPT, REM)])

        plsc.subcore_barrier()

        for d in range(D):
            pltpu.async_copy(h_hbm.at[src_v.at[d]], rows[d], gsem[d])

        @pl.loop(0, CH, step=B)
        def _(j):
            for b in range(B):
                jj = j + b
                bn = (b + D) % B

                @pl.when(jnp.logical_and(lax.rem(jj, 8) == 0, jj + 8 < CH))
                def _():
                    pltpu.async_copy(
                        src2d.at[pl.ds(_al8(w * CH + jj + 8), 8)],
                        src_v.at[pl.ds(_al8(lax.rem(jj // 8 + 1, 2) * 8), 8)],
                        rsem)

                @pl.when(jnp.logical_and(lax.rem(jj, 8) == 6, jj + D < CH))
                def _():
                    pltpu.make_async_copy(
                        src2d.at[pl.ds(_al8(w * CH), 8)],
                        src_v.at[pl.ds(0, 8)], rsem).wait()

                @pl.when(jj + D < CH)
                def _():
                    @pl.when(jj + D >= B)
                    def _():
                        pltpu.make_async_copy(
                            rows[bn], acc.at[dst_v.at[0]], ssem[bn]).wait()

                    pltpu.async_copy(
                        h_hbm.at[src_v.at[lax.rem(jj + D, 16)]],
                        rows[bn], gsem[bn])

                pltpu.make_async_copy(
                    h_hbm.at[src_v.at[lax.rem(jj, 16)]],
                    rows[b], gsem[b]).wait()
                pltpu.async_copy(rows[b], acc.at[dst_v.at[jj]], ssem[b],
                                 add=True)

        for cc in range(CH - B, CH):
            pltpu.make_async_copy(rows[cc % B], acc.at[dst_v.at[0]],
                                  ssem[cc % B]).wait()

        plsc.subcore_barrier()
        sl = pl.ds(_al8(s * RPT), RPT)
        pltpu.sync_copy(acc.at[sl], part_out.at[c].at[sl])

        @pl.when(s == 0)
        def _():
            rl = pl.ds(NS * RPT, REM)
            pltpu.sync_copy(acc.at[rl], part_out.at[c].at[rl])

    return sc_scatter


def _make_sc_scatter128_b3():
    F, B, D = 128, 3, 1

    def r_src(cc):
        return lax.rem(cc // 4, 2) * 8 + 2 * lax.rem(cc, 4)

    @functools.partial(
        pl.kernel,
        out_type=jax.ShapeDtypeStruct((NC, N_NODES, F), jnp.float32),
        mesh=_mesh(),
        compiler_params=pltpu.CompilerParams(use_tc_tiling_on_sc=False),
        scratch_types=(
            [pltpu.VMEM((16, K), jnp.int32)]
            + [pltpu.VMEM((K, F), jnp.float32) for _ in range(B)]
            + [pltpu.VMEM_SHARED((N_NODES, F), jnp.float32)]
            + [pltpu.SemaphoreType.DMA for _ in range(2 * B + 1)]
        ),
    )
    def sc_scatter(sd2d, h_hbm, part_out, idx_v, *bufs_sems):
        rows = bufs_sems[:B]
        acc = bufs_sems[B]
        gsem = bufs_sems[B + 1:2 * B + 1]
        ssem = bufs_sems[2 * B + 1:3 * B + 1]
        rsem = bufs_sems[3 * B + 1]
        c = lax.axis_index("c")
        s = lax.axis_index("s")
        w = _wid()
        pltpu.sync_copy(sd2d.at[pl.ds(_al8(2 * w * CH), 16)], idx_v)

        z16 = jnp.zeros((16,), jnp.float32)

        @pl.loop(0, K)
        def _(i):
            for g in range(F // 16):
                rows[0][i, pl.ds(16 * g, 16)] = z16

        for m in range(RPT // 104):
            pltpu.async_copy(rows[0].at[pl.ds(0, 104)],
                             acc.at[pl.ds(_al8(s * RPT + m * 104), 104)],
                             ssem[0])
        for m in range(RPT // 104):
            pltpu.make_async_copy(rows[0].at[pl.ds(0, 104)],
                                  acc.at[pl.ds(_al8(s * RPT + m * 104), 104)],
                                  ssem[0]).wait()

        @pl.when(s == 0)
        def _():
            pltpu.sync_copy(rows[0].at[pl.ds(0, REM)],
                            acc.at[pl.ds(NS * RPT, REM)])

        plsc.subcore_barrier()

        pltpu.async_copy(h_hbm.at[idx_v.at[0]], rows[0], gsem[0])

        @pl.loop(0, CH - CH % B, step=B)
        def _(j):
            for b in range(B):
                jj = j + b
                bn = (b + D) % B

                @pl.when(jnp.logical_and(
                        jnp.logical_and(lax.rem(jj, 4) == 2, jj >= 6),
                        (jj // 4 + 1) * 4 < CH))
                def _():
                    pltpu.async_copy(
                        sd2d.at[pl.ds(_al8(2 * (w * CH + (jj // 4 + 1) * 4)),
                                      8)],
                        idx_v.at[pl.ds(_al8(lax.rem(jj // 4 + 1, 2) * 8), 8)],
                        rsem)

                @pl.when(jnp.logical_and(
                        jnp.logical_and(lax.rem(jj, 4) == 3, jj >= 7),
                        jj + 1 < CH))
                def _():
                    pltpu.make_async_copy(
                        sd2d.at[pl.ds(_al8(2 * w * CH), 8)],
                        idx_v.at[pl.ds(0, 8)], rsem).wait()

                @pl.when(jj + D < CH)
                def _():
                    @pl.when(jj + D >= B)
                    def _():
                        pltpu.make_async_copy(
                            rows[bn], acc.at[idx_v.at[1]], ssem[bn]).wait()

                    pltpu.async_copy(
                        h_hbm.at[idx_v.at[r_src(jj + D)]], rows[bn],
                        gsem[bn])

                pltpu.make_async_copy(
                    h_hbm.at[idx_v.at[r_src(jj)]], rows[b], gsem[b]).wait()
                pltpu.async_copy(rows[b], acc.at[idx_v.at[r_src(jj) + 1]],
                                 ssem[b], add=True)

        pltpu.make_async_copy(rows[1], acc.at[idx_v.at[1]], ssem[1]).wait()
        pltpu.async_copy(h_hbm.at[idx_v.at[14]], rows[1], gsem[1])
        pltpu.make_async_copy(h_hbm.at[idx_v.at[12]], rows[0], gsem[0]).wait()
        pltpu.async_copy(rows[0], acc.at[idx_v.at[13]], ssem[0], add=True)
        pltpu.make_async_copy(h_hbm.at[idx_v.at[14]], rows[1], gsem[1]).wait()
        pltpu.async_copy(rows[1], acc.at[idx_v.at[15]], ssem[1], add=True)
        for b in range(B):
            pltpu.make_async_copy(rows[b], acc.at[idx_v.at[1]],
                                  ssem[b]).wait()

        plsc.subcore_barrier()
        sl = pl.ds(_al8(s * RPT), RPT)
        pltpu.sync_copy(acc.at[sl], part_out.at[c].at[sl])

        @pl.when(s == 0)
        def _():
            rl = pl.ds(NS * RPT, REM)
            pltpu.sync_copy(acc.at[rl], part_out.at[c].at[rl])

    return sc_scatter


_sc_scatter128 = _make_sc_scatter128_b3()
_sc_scatter64 = _make_sc_scatter_nbuf(64, 4, 2)


BN = 1000


def _tc_mm0(features, W0):
    def body(x_ref, w_ref, h_ref):
        h_ref[...] = jnp.dot(x_ref[...], w_ref[...],
                             preferred_element_type=jnp.float32)

    return pl.pallas_call(
        body,
        grid=(N_NODES // BN,),
        in_specs=[
            pl.BlockSpec((BN, 128), lambda i: (i, 0)),
            pl.BlockSpec((128, 128), lambda i: (0, 0)),
        ],
        out_specs=pl.BlockSpec((BN, 128), lambda i: (i, 0)),
        out_shape=jax.ShapeDtypeStruct((N_NODES, 128), jnp.float32),
    )(features, W0)


def _tc_norms(h0, dsrc_p, ddst_p):
    def body(h0_ref, ds_ref, dd_ref, h_ref, ns_ref, nd_ref):
        ds = ds_ref[0, :, 0:1] + ds_ref[1, :, 0:1]
        dd = dd_ref[0, :, 0:1] + dd_ref[1, :, 0:1]
        ns = lax.rsqrt(jnp.maximum(ds, 1.0))
        nd = lax.rsqrt(jnp.maximum(dd, 1.0))
        ns_ref[...] = ns
        nd_ref[...] = nd
        h_ref[...] = h0_ref[...] * ns

    return pl.pallas_call(
        body,
        grid=(N_NODES // BN,),
        in_specs=[
            pl.BlockSpec((BN, 128), lambda i: (i, 0)),
            pl.BlockSpec((NC, BN, 16), lambda i: (0, i, 0)),
            pl.BlockSpec((NC, BN, 16), lambda i: (0, i, 0)),
        ],
        out_specs=[
            pl.BlockSpec((BN, 128), lambda i: (i, 0)),
            pl.BlockSpec((BN, 1), lambda i: (i, 0)),
            pl.BlockSpec((BN, 1), lambda i: (i, 0)),
        ],
        out_shape=[
            jax.ShapeDtypeStruct((N_NODES, 128), jnp.float32),
            jax.ShapeDtypeStruct((N_NODES, 1), jnp.float32),
            jax.ShapeDtypeStruct((N_NODES, 1), jnp.float32),
        ],
    )(h0, dsrc_p, ddst_p)


def _make_tc_combine(F, FO, act, emit_agg, with_mm):
    def body(*refs):
        if with_mm:
            p_ref, nd_ref, b_ref, ns_ref, w_ref = refs[:5]
            orefs = refs[5:]
        else:
            p_ref, nd_ref, b_ref = refs[:3]
            orefs = refs[3:]
        agg = (p_ref[0] + p_ref[1]) * nd_ref[...] + b_ref[...]
        if act:
            agg = jnp.maximum(agg, 0.0)
        oi = 0
        if emit_agg:
            orefs[oi][...] = agg
            oi += 1
        if with_mm:
            orefs[oi][...] = jnp.dot(
                agg, w_ref[...], preferred_element_type=jnp.float32) * ns_ref[...]

    in_specs = [
        pl.BlockSpec((NC, BN, F), lambda i: (0, i, 0)),
        pl.BlockSpec((BN, 1), lambda i: (i, 0)),
        pl.BlockSpec((1, F), lambda i: (0, 0)),
    ]
    out_specs, out_shape = [], []
    if with_mm:
        in_specs += [
            pl.BlockSpec((BN, 1), lambda i: (i, 0)),
            pl.BlockSpec((F, FO), lambda i: (0, 0)),
        ]
    if emit_agg:
        out_specs.append(pl.BlockSpec((BN, F), lambda i: (i, 0)))
        out_shape.append(jax.ShapeDtypeStruct((N_NODES, F), jnp.float32))
    if with_mm:
        out_specs.append(pl.BlockSpec((BN, FO), lambda i: (i, 0)))
        out_shape.append(jax.ShapeDtypeStruct((N_NODES, FO), jnp.float32))

    kern = pl.pallas_call(
        body, grid=(N_NODES // BN,),
        in_specs=in_specs, out_specs=out_specs, out_shape=out_shape)

    def run(p, ndst, b2d, nsrc=None, W=None):
        args = (p, ndst, b2d) + ((nsrc, W) if with_mm else ())
        out = kern(*args)
        return out if len(out) > 1 else out[0]
    return run


_tc_mid_128_128 = _make_tc_combine(128, 128, act=True, emit_agg=False, with_mm=True)
_tc_mid_128_64 = _make_tc_combine(128, 64, act=True, emit_agg=False, with_mm=True)
_tc_aspect = _make_tc_combine(64, 64, act=False, emit_agg=True, with_mm=True)
_tc_last = _make_tc_combine(64, 0, act=False, emit_agg=True, with_mm=False)


def kernel(features, edge_index, W0, b0, W1, b1, W2, b2, W3, b3):
    src2d = edge_index[0].reshape(NW * CH, K)
    dst2d = edge_index[1].reshape(NW * CH, K)
    sd2d = jnp.stack([src2d, dst2d], axis=1).reshape(2 * NW * CH, K)

    h0 = _tc_mm0(features, W0)
    dsrc_p, ddst_p = _sc_degrees(src2d, dst2d)
    h0s, nsrc, ndst = _tc_norms(h0, dsrc_p, ddst_p)

    p1 = _sc_scatter128(sd2d, h0s)
    h1s = _tc_mid_128_128(p1, ndst, b0.reshape(1, -1), nsrc, W1)

    p2 = _sc_scatter128(sd2d, h1s)
    h2s = _tc_mid_128_64(p2, ndst, b1.reshape(1, -1), nsrc, W2)

    p3 = _sc_scatter64(src2d, dst2d, h2s)
    aspect, h3s = _tc_aspect(p3, ndst, b2.reshape(1, -1), nsrc, W3)

    p4 = _sc_scatter64(src2d, dst2d, h3s)
    out = _tc_last(p4, ndst, b3.reshape(1, -1))

    return (aspect, out)
